# baseline (speedup 1.0000x reference)
import functools

import jax
import jax.numpy as jnp
from jax import lax
from jax.experimental import pallas as pl
from jax.experimental.pallas import tpu as pltpu
from jax.experimental.pallas import tpu_sc as plsc

_N = 10000
_E = 320000
_D = 128

_NC = 2
_NS = 16
_CHUNK = 128
_RPT = 320
_NROWS = _NS * _RPT
_GRP = 16
_TOT = _NROWS * _CHUNK
_PAD_ROW = _N
_DUMP = 640
_ACC_ROWS = _NS * _DUMP


def _agg_sc(h, edges3, zeros_acc):
  mesh = plsc.VectorSubcoreMesh(core_axis_name="c", subcore_axis_name="s",
                                num_cores=1)

  @functools.partial(
      pl.kernel,
      out_type=jax.ShapeDtypeStruct((1, _ACC_ROWS, _D), jnp.float32),
      mesh=mesh,
      scratch_types=[
          pltpu.VMEM((_GRP, _CHUNK), jnp.int32),
          pltpu.VMEM((_GRP, _CHUNK), jnp.int32),
          pltpu.VMEM((_CHUNK, _D), jnp.float32),
          pltpu.VMEM((_CHUNK, _D), jnp.float32),
          pltpu.SemaphoreType.DMA,
          pltpu.SemaphoreType.DMA,
          pltpu.SemaphoreType.DMA,
          pltpu.SemaphoreType.DMA,
          pltpu.VMEM_SHARED((_ACC_ROWS, _D), jnp.float32),
      ],
  )
  def k(h_hbm, e_hbm, z_hbm, out_hbm, src_v, dst_v, rows0, rows1,
        gs0, gs1, ss0, ss1, acc_sh):
    s = lax.axis_index("s")
    row_base = s * _RPT
    ngroups = _RPT // _GRP

    @pl.when(s == 0)
    def _():
      pltpu.sync_copy(z_hbm, acc_sh)

    plsc.subcore_barrier()

    def wait(sem, buf):
      pltpu.make_async_copy(h_hbm.at[src_v.at[0]], buf, sem).wait()

    def group(g, carry):
      pltpu.sync_copy(e_hbm.at[0].at[pl.ds(row_base + g * _GRP, _GRP)],
                      src_v)
      pltpu.sync_copy(e_hbm.at[1].at[pl.ds(row_base + g * _GRP, _GRP)],
                      dst_v)

      pltpu.async_copy(h_hbm.at[src_v.at[0]], rows0, gs0)
      pltpu.async_copy(h_hbm.at[src_v.at[1]], rows1, gs1)

      def pair(i, c2):
        j0 = 2 * i
        wait(gs0, rows0)
        pltpu.async_copy(rows0, acc_sh.at[dst_v.at[j0]], ss0, add=True)
        wait(gs1, rows1)
        pltpu.async_copy(rows1, acc_sh.at[dst_v.at[j0 + 1]], ss1, add=True)
        wait(ss0, rows0)
        pltpu.async_copy(h_hbm.at[src_v.at[j0 + 2]], rows0, gs0)
        wait(ss1, rows1)
        pltpu.async_copy(h_hbm.at[src_v.at[j0 + 3]], rows1, gs1)
        return c2

      lax.fori_loop(0, _GRP // 2 - 1, pair, 0, unroll=False)

      wait(gs0, rows0)
      pltpu.async_copy(rows0, acc_sh.at[dst_v.at[_GRP - 2]], ss0, add=True)
      wait(gs1, rows1)
      pltpu.async_copy(rows1, acc_sh.at[dst_v.at[_GRP - 1]], ss1, add=True)
      wait(ss0, rows0)
      wait(ss1, rows1)
      return carry

    lax.fori_loop(0, ngroups, group, 0, unroll=False)
    plsc.subcore_barrier()

    pltpu.sync_copy(acc_sh.at[pl.ds(s * _DUMP, _DUMP)],
                    out_hbm.at[0].at[pl.ds(s * _DUMP, _DUMP)])

  return k(h, edges3, zeros_acc)


def _matmul_t(a, w):
  return lax.dot_general(a, w, (((1,), (1,)), ((), ())),
                         preferred_element_type=jnp.float32)


def _bn_relu(h, g, b):
  m = jnp.mean(h, axis=0, keepdims=True)
  v = jnp.mean((h - m) ** 2, axis=0, keepdims=True)
  return jnp.maximum(g * (h - m) * lax.rsqrt(v + 1e-5) + b, 0.0)


def _tc_in(x, W_in, b_in, g_in, beta_in):
  def body(x_ref, w_ref, b_ref, g_ref, be_ref, o_ref):
    h = _matmul_t(x_ref[...], w_ref[...]) + b_ref[...]
    o_ref[...] = _bn_relu(h, g_ref[...], be_ref[...])

  return pl.pallas_call(
      body,
      out_shape=jax.ShapeDtypeStruct((_N, _D), jnp.float32),
  )(x, W_in, b_in, g_in, beta_in)


def _tc_gin(h, acc, eps, Wa, ba, Wb, bb, g, beta):
  def body(h_ref, acc_ref, eps_ref, wa_ref, ba_ref, wb_ref, bb_ref, g_ref,
           be_ref, o_ref):
    z = (2.0 + eps_ref[0, 0]) * h_ref[...] + acc_ref[0, :_N]
    t = jnp.maximum(_matmul_t(z, wa_ref[...]) + ba_ref[...], 0.0)
    u = _matmul_t(t, wb_ref[...]) + bb_ref[...]
    o_ref[...] = _bn_relu(u, g_ref[...], be_ref[...])

  return pl.pallas_call(
      body,
      out_shape=jax.ShapeDtypeStruct((_N, _D), jnp.float32),
  )(h, acc, eps, Wa, ba, Wb, bb, g, beta)


def _tc_head(h, W_head, b_head):
  def body(h_ref, w_ref, b_ref, o_ref):
    o_ref[...] = _matmul_t(h_ref[...], w_ref[...]) + b_ref[...]

  return pl.pallas_call(
      body,
      out_shape=jax.ShapeDtypeStruct((_N, _D), jnp.float32),
  )(h, W_head, b_head)


def kernel(x, edge_index, eps1, eps2, W_in, b_in, g_in, beta_in, W1a, b1a,
           W1b, b1b, g1, beta1, W2a, b2a, W2b, b2b, g2, beta2, W_head,
           b_head):
  ei = edge_index.astype(jnp.int32)
  pad = _TOT - 2 * _E
  src = jnp.concatenate([ei[0], ei[1], jnp.zeros((pad,), jnp.int32)])
  pad_dst = _PAD_ROW + (jnp.arange(pad, dtype=jnp.int32) % (_ACC_ROWS - _N))
  dst = jnp.concatenate([ei[1], ei[0], pad_dst])
  edges3 = jnp.stack([src, dst]).reshape(2, _NROWS, _CHUNK)
  zeros_acc = jnp.zeros((_ACC_ROWS, _D), jnp.float32)

  r = lambda v: v.reshape(1, _D)
  x0 = _tc_in(x, W_in, r(b_in), r(g_in), r(beta_in))
  acc1 = _agg_sc(x0, edges3, zeros_acc)
  h1 = _tc_gin(x0, acc1, eps1.reshape(1, 1), W1a, r(b1a), W1b, r(b1b),
               r(g1), r(beta1))
  acc2 = _agg_sc(h1, edges3, zeros_acc)
  h2 = _tc_gin(h1, acc2, eps2.reshape(1, 1), W2a, r(b2a), W2b, r(b2b),
               r(g2), r(beta2))
  return _tc_head(h2, W_head, r(b_head))


# speedup vs baseline: 1.4887x; 1.4887x over previous
import functools

import jax
import jax.numpy as jnp
from jax import lax
from jax.experimental import pallas as pl
from jax.experimental.pallas import tpu as pltpu
from jax.experimental.pallas import tpu_sc as plsc

_N = 10000
_E = 320000
_D = 128

_NC = 2
_NS = 16
_CHUNK = 128
_R0 = 256
_R1 = 64
_NROWS = _NS * (_R0 + _R1)
_GRP = 16
_TOT = _NROWS * _CHUNK
_PAD_ROW = _N
_DUMP = 640
_ACC_ROWS = _NS * _DUMP


def _agg_sc(h, edges3, zeros_acc):
  mesh = plsc.VectorSubcoreMesh(core_axis_name="c", subcore_axis_name="s")

  @functools.partial(
      pl.kernel,
      out_type=jax.ShapeDtypeStruct((_NC, _ACC_ROWS, _D), jnp.float32),
      mesh=mesh,
      scratch_types=[
          pltpu.VMEM((_GRP, _CHUNK), jnp.int32),
          pltpu.VMEM((_GRP, _CHUNK), jnp.int32),
          pltpu.VMEM((_CHUNK, _D), jnp.float32),
          pltpu.VMEM((_CHUNK, _D), jnp.float32),
          pltpu.SemaphoreType.DMA,
          pltpu.SemaphoreType.DMA,
          pltpu.SemaphoreType.DMA,
          pltpu.SemaphoreType.DMA,
          pltpu.VMEM_SHARED((_ACC_ROWS, _D), jnp.float32),
      ],
  )
  def k(h_hbm, e_hbm, z_hbm, out_hbm, src_v, dst_v, rows0, rows1,
        gs0, gs1, ss0, ss1, acc_sh):
    c = lax.axis_index("c")
    s = lax.axis_index("s")
    row_base = jnp.where(c == 0, s * _R0, _NS * _R0 + s * _R1)
    ngroups = jnp.where(c == 0, _R0 // _GRP, _R1 // _GRP)

    @pl.when(s == 0)
    def _():
      pltpu.sync_copy(z_hbm, acc_sh)

    plsc.subcore_barrier()

    def wait(sem, buf):
      pltpu.make_async_copy(h_hbm.at[src_v.at[0]], buf, sem).wait()

    def group(g, carry):
      pltpu.sync_copy(e_hbm.at[0].at[pl.ds(row_base + g * _GRP, _GRP)],
                      src_v)
      pltpu.sync_copy(e_hbm.at[1].at[pl.ds(row_base + g * _GRP, _GRP)],
                      dst_v)

      pltpu.async_copy(h_hbm.at[src_v.at[0]], rows0, gs0)
      pltpu.async_copy(h_hbm.at[src_v.at[1]], rows1, gs1)

      def pair(i, c2):
        j0 = 2 * i
        wait(gs0, rows0)
        pltpu.async_copy(rows0, acc_sh.at[dst_v.at[j0]], ss0, add=True)
        wait(gs1, rows1)
        pltpu.async_copy(rows1, acc_sh.at[dst_v.at[j0 + 1]], ss1, add=True)
        wait(ss0, rows0)
        pltpu.async_copy(h_hbm.at[src_v.at[j0 + 2]], rows0, gs0)
        wait(ss1, rows1)
        pltpu.async_copy(h_hbm.at[src_v.at[j0 + 3]], rows1, gs1)
        return c2

      lax.fori_loop(0, _GRP // 2 - 1, pair, 0, unroll=False)

      wait(gs0, rows0)
      pltpu.async_copy(rows0, acc_sh.at[dst_v.at[_GRP - 2]], ss0, add=True)
      wait(gs1, rows1)
      pltpu.async_copy(rows1, acc_sh.at[dst_v.at[_GRP - 1]], ss1, add=True)
      wait(ss0, rows0)
      wait(ss1, rows1)
      return carry

    lax.fori_loop(0, ngroups, group, 0, unroll=False)
    plsc.subcore_barrier()

    pltpu.sync_copy(acc_sh.at[pl.ds(s * _DUMP, _DUMP)],
                    out_hbm.at[c].at[pl.ds(s * _DUMP, _DUMP)])

  return k(h, edges3, zeros_acc)


def _matmul_t(a, w):
  return lax.dot_general(a, w, (((1,), (1,)), ((), ())),
                         preferred_element_type=jnp.float32)


def _bn_relu(h, g, b):
  m = jnp.mean(h, axis=0, keepdims=True)
  v = jnp.mean((h - m) ** 2, axis=0, keepdims=True)
  return jnp.maximum(g * (h - m) * lax.rsqrt(v + 1e-5) + b, 0.0)


def _tc_in(x, W_in, b_in, g_in, beta_in):
  def body(x_ref, w_ref, b_ref, g_ref, be_ref, o_ref):
    h = _matmul_t(x_ref[...], w_ref[...]) + b_ref[...]
    o_ref[...] = _bn_relu(h, g_ref[...], be_ref[...])

  return pl.pallas_call(
      body,
      out_shape=jax.ShapeDtypeStruct((_N, _D), jnp.float32),
  )(x, W_in, b_in, g_in, beta_in)


def _tc_gin(h, acc, eps, Wa, ba, Wb, bb, g, beta):
  def body(h_ref, acc_ref, eps_ref, wa_ref, ba_ref, wb_ref, bb_ref, g_ref,
           be_ref, o_ref):
    z = ((2.0 + eps_ref[0, 0]) * h_ref[...] + acc_ref[0, :_N]
         + acc_ref[1, :_N])
    t = jnp.maximum(_matmul_t(z, wa_ref[...]) + ba_ref[...], 0.0)
    u = _matmul_t(t, wb_ref[...]) + bb_ref[...]
    o_ref[...] = _bn_relu(u, g_ref[...], be_ref[...])

  return pl.pallas_call(
      body,
      out_shape=jax.ShapeDtypeStruct((_N, _D), jnp.float32),
  )(h, acc, eps, Wa, ba, Wb, bb, g, beta)


def _tc_head(h, W_head, b_head):
  def body(h_ref, w_ref, b_ref, o_ref):
    o_ref[...] = _matmul_t(h_ref[...], w_ref[...]) + b_ref[...]

  return pl.pallas_call(
      body,
      out_shape=jax.ShapeDtypeStruct((_N, _D), jnp.float32),
  )(h, W_head, b_head)


def kernel(x, edge_index, eps1, eps2, W_in, b_in, g_in, beta_in, W1a, b1a,
           W1b, b1b, g1, beta1, W2a, b2a, W2b, b2b, g2, beta2, W_head,
           b_head):
  ei = edge_index.astype(jnp.int32)
  pad = _TOT - 2 * _E
  src = jnp.concatenate([ei[0], ei[1], jnp.zeros((pad,), jnp.int32)])
  pad_dst = _PAD_ROW + (jnp.arange(pad, dtype=jnp.int32) % (_ACC_ROWS - _N))
  dst = jnp.concatenate([ei[1], ei[0], pad_dst])
  edges3 = jnp.stack([src, dst]).reshape(2, _NROWS, _CHUNK)
  zeros_acc = jnp.zeros((_ACC_ROWS, _D), jnp.float32)

  r = lambda v: v.reshape(1, _D)
  x0 = _tc_in(x, W_in, r(b_in), r(g_in), r(beta_in))
  acc1 = _agg_sc(x0, edges3, zeros_acc)
  h1 = _tc_gin(x0, acc1, eps1.reshape(1, 1), W1a, r(b1a), W1b, r(b1b),
               r(g1), r(beta1))
  acc2 = _agg_sc(h1, edges3, zeros_acc)
  h2 = _tc_gin(h1, acc2, eps2.reshape(1, 1), W2a, r(b2a), W2b, r(b2b),
               r(g2), r(beta2))
  return _tc_head(h2, W_head, r(b_head))


# speedup vs baseline: 1.6365x; 1.0993x over previous
import functools

import jax
import jax.numpy as jnp
from jax import lax
from jax.experimental import pallas as pl
from jax.experimental.pallas import tpu as pltpu
from jax.experimental.pallas import tpu_sc as plsc

_N = 10000
_E = 320000
_D = 128

_NC = 2
_NS = 16
_CHUNK = 128
_R0 = 288
_R1 = 32
_NROWS = _NS * (_R0 + _R1)
_GRP = 16
_TOT = _NROWS * _CHUNK
_PAD_ROW = _N
_DUMP = 640
_ACC_ROWS = _NS * _DUMP


def _agg_sc(h, edges3, zeros_acc):
  mesh = plsc.VectorSubcoreMesh(core_axis_name="c", subcore_axis_name="s")

  @functools.partial(
      pl.kernel,
      out_type=jax.ShapeDtypeStruct((_NC, _ACC_ROWS, _D), jnp.float32),
      mesh=mesh,
      scratch_types=[
          pltpu.VMEM((_GRP, _CHUNK), jnp.int32),
          pltpu.VMEM((_GRP, _CHUNK), jnp.int32),
          pltpu.VMEM((_CHUNK, _D), jnp.float32),
          pltpu.VMEM((_CHUNK, _D), jnp.float32),
          pltpu.SemaphoreType.DMA,
          pltpu.SemaphoreType.DMA,
          pltpu.SemaphoreType.DMA,
          pltpu.SemaphoreType.DMA,
          pltpu.VMEM_SHARED((_ACC_ROWS, _D), jnp.float32),
      ],
  )
  def k(h_hbm, e_hbm, z_hbm, out_hbm, src_v, dst_v, rows0, rows1,
        gs0, gs1, ss0, ss1, acc_sh):
    c = lax.axis_index("c")
    s = lax.axis_index("s")
    row_base = jnp.where(c == 0, s * _R0, _NS * _R0 + s * _R1)
    ngroups = jnp.where(c == 0, _R0 // _GRP, _R1 // _GRP)

    @pl.when(s == 0)
    def _():
      pltpu.sync_copy(z_hbm, acc_sh)

    plsc.subcore_barrier()

    def wait(sem, buf):
      pltpu.make_async_copy(h_hbm.at[src_v.at[0]], buf, sem).wait()

    def group(g, carry):
      pltpu.sync_copy(e_hbm.at[0].at[pl.ds(row_base + g * _GRP, _GRP)],
                      src_v)
      pltpu.sync_copy(e_hbm.at[1].at[pl.ds(row_base + g * _GRP, _GRP)],
                      dst_v)

      pltpu.async_copy(h_hbm.at[src_v.at[0]], rows0, gs0)
      pltpu.async_copy(h_hbm.at[src_v.at[1]], rows1, gs1)

      def pair(i, c2):
        j0 = 2 * i
        wait(gs0, rows0)
        pltpu.async_copy(rows0, acc_sh.at[dst_v.at[j0]], ss0, add=True)
        wait(gs1, rows1)
        pltpu.async_copy(rows1, acc_sh.at[dst_v.at[j0 + 1]], ss1, add=True)
        wait(ss0, rows0)
        pltpu.async_copy(h_hbm.at[src_v.at[j0 + 2]], rows0, gs0)
        wait(ss1, rows1)
        pltpu.async_copy(h_hbm.at[src_v.at[j0 + 3]], rows1, gs1)
        return c2

      lax.fori_loop(0, _GRP // 2 - 1, pair, 0, unroll=False)

      wait(gs0, rows0)
      pltpu.async_copy(rows0, acc_sh.at[dst_v.at[_GRP - 2]], ss0, add=True)
      wait(gs1, rows1)
      pltpu.async_copy(rows1, acc_sh.at[dst_v.at[_GRP - 1]], ss1, add=True)
      wait(ss0, rows0)
      wait(ss1, rows1)
      return carry

    lax.fori_loop(0, ngroups, group, 0, unroll=False)
    plsc.subcore_barrier()

    pltpu.sync_copy(acc_sh.at[pl.ds(s * _DUMP, _DUMP)],
                    out_hbm.at[c].at[pl.ds(s * _DUMP, _DUMP)])

  return k(h, edges3, zeros_acc)


def _matmul_t(a, w):
  return lax.dot_general(a, w, (((1,), (1,)), ((), ())),
                         preferred_element_type=jnp.float32)


def _bn_relu(h, g, b):
  m = jnp.mean(h, axis=0, keepdims=True)
  v = jnp.mean((h - m) ** 2, axis=0, keepdims=True)
  return jnp.maximum(g * (h - m) * lax.rsqrt(v + 1e-5) + b, 0.0)


def _tc_in(x, W_in, b_in, g_in, beta_in):
  def body(x_ref, w_ref, b_ref, g_ref, be_ref, o_ref):
    h = _matmul_t(x_ref[...], w_ref[...]) + b_ref[...]
    o_ref[...] = _bn_relu(h, g_ref[...], be_ref[...])

  return pl.pallas_call(
      body,
      out_shape=jax.ShapeDtypeStruct((_N, _D), jnp.float32),
  )(x, W_in, b_in, g_in, beta_in)


def _tc_gin(h, acc, eps, Wa, ba, Wb, bb, g, beta):
  def body(h_ref, acc_ref, eps_ref, wa_ref, ba_ref, wb_ref, bb_ref, g_ref,
           be_ref, o_ref):
    z = ((2.0 + eps_ref[0, 0]) * h_ref[...] + acc_ref[0, :_N]
         + acc_ref[1, :_N])
    t = jnp.maximum(_matmul_t(z, wa_ref[...]) + ba_ref[...], 0.0)
    u = _matmul_t(t, wb_ref[...]) + bb_ref[...]
    o_ref[...] = _bn_relu(u, g_ref[...], be_ref[...])

  return pl.pallas_call(
      body,
      out_shape=jax.ShapeDtypeStruct((_N, _D), jnp.float32),
  )(h, acc, eps, Wa, ba, Wb, bb, g, beta)


def _tc_head(h, W_head, b_head):
  def body(h_ref, w_ref, b_ref, o_ref):
    o_ref[...] = _matmul_t(h_ref[...], w_ref[...]) + b_ref[...]

  return pl.pallas_call(
      body,
      out_shape=jax.ShapeDtypeStruct((_N, _D), jnp.float32),
  )(h, W_head, b_head)


def kernel(x, edge_index, eps1, eps2, W_in, b_in, g_in, beta_in, W1a, b1a,
           W1b, b1b, g1, beta1, W2a, b2a, W2b, b2b, g2, beta2, W_head,
           b_head):
  ei = edge_index.astype(jnp.int32)
  pad = _TOT - 2 * _E
  src = jnp.concatenate([ei[0], ei[1], jnp.zeros((pad,), jnp.int32)])
  pad_dst = _PAD_ROW + (jnp.arange(pad, dtype=jnp.int32) % (_ACC_ROWS - _N))
  dst = jnp.concatenate([ei[1], ei[0], pad_dst])
  edges3 = jnp.stack([src, dst]).reshape(2, _NROWS, _CHUNK)
  zeros_acc = jnp.zeros((_ACC_ROWS, _D), jnp.float32)

  r = lambda v: v.reshape(1, _D)
  x0 = _tc_in(x, W_in, r(b_in), r(g_in), r(beta_in))
  acc1 = _agg_sc(x0, edges3, zeros_acc)
  h1 = _tc_gin(x0, acc1, eps1.reshape(1, 1), W1a, r(b1a), W1b, r(b1b),
               r(g1), r(beta1))
  acc2 = _agg_sc(h1, edges3, zeros_acc)
  h2 = _tc_gin(h1, acc2, eps2.reshape(1, 1), W2a, r(b2a), W2b, r(b2b),
               r(g2), r(beta2))
  return _tc_head(h2, W_head, r(b_head))


# speedup vs baseline: 1.6436x; 1.0043x over previous
import functools

import jax
import jax.numpy as jnp
from jax import lax
from jax.experimental import pallas as pl
from jax.experimental.pallas import tpu as pltpu
from jax.experimental.pallas import tpu_sc as plsc

_N = 10000
_E = 320000
_D = 128

_NC = 2
_NS = 16
_CHUNK = 128
_R0 = 304
_R1 = 16
_NROWS = _NS * (_R0 + _R1)
_GRP = 16
_TOT = _NROWS * _CHUNK
_PAD_ROW = _N
_DUMP = 640
_ACC_ROWS = _NS * _DUMP


def _agg_sc(h, edges3, zeros_acc):
  mesh = plsc.VectorSubcoreMesh(core_axis_name="c", subcore_axis_name="s")

  @functools.partial(
      pl.kernel,
      out_type=jax.ShapeDtypeStruct((_NC, _ACC_ROWS, _D), jnp.float32),
      mesh=mesh,
      scratch_types=[
          pltpu.VMEM((_GRP, _CHUNK), jnp.int32),
          pltpu.VMEM((_GRP, _CHUNK), jnp.int32),
          pltpu.VMEM((_CHUNK, _D), jnp.float32),
          pltpu.VMEM((_CHUNK, _D), jnp.float32),
          pltpu.SemaphoreType.DMA,
          pltpu.SemaphoreType.DMA,
          pltpu.SemaphoreType.DMA,
          pltpu.SemaphoreType.DMA,
          pltpu.VMEM_SHARED((_ACC_ROWS, _D), jnp.float32),
      ],
  )
  def k(h_hbm, e_hbm, z_hbm, out_hbm, src_v, dst_v, rows0, rows1,
        gs0, gs1, ss0, ss1, acc_sh):
    c = lax.axis_index("c")
    s = lax.axis_index("s")
    row_base = jnp.where(c == 0, s * _R0, _NS * _R0 + s * _R1)
    ngroups = jnp.where(c == 0, _R0 // _GRP, _R1 // _GRP)

    @pl.when(s == 0)
    def _():
      pltpu.sync_copy(z_hbm, acc_sh)

    plsc.subcore_barrier()

    def wait(sem, buf):
      pltpu.make_async_copy(h_hbm.at[src_v.at[0]], buf, sem).wait()

    def group(g, carry):
      pltpu.sync_copy(e_hbm.at[0].at[pl.ds(row_base + g * _GRP, _GRP)],
                      src_v)
      pltpu.sync_copy(e_hbm.at[1].at[pl.ds(row_base + g * _GRP, _GRP)],
                      dst_v)

      pltpu.async_copy(h_hbm.at[src_v.at[0]], rows0, gs0)
      pltpu.async_copy(h_hbm.at[src_v.at[1]], rows1, gs1)

      def pair(i, c2):
        j0 = 2 * i
        wait(gs0, rows0)
        pltpu.async_copy(rows0, acc_sh.at[dst_v.at[j0]], ss0, add=True)
        wait(gs1, rows1)
        pltpu.async_copy(rows1, acc_sh.at[dst_v.at[j0 + 1]], ss1, add=True)
        wait(ss0, rows0)
        pltpu.async_copy(h_hbm.at[src_v.at[j0 + 2]], rows0, gs0)
        wait(ss1, rows1)
        pltpu.async_copy(h_hbm.at[src_v.at[j0 + 3]], rows1, gs1)
        return c2

      lax.fori_loop(0, _GRP // 2 - 1, pair, 0, unroll=False)

      wait(gs0, rows0)
      pltpu.async_copy(rows0, acc_sh.at[dst_v.at[_GRP - 2]], ss0, add=True)
      wait(gs1, rows1)
      pltpu.async_copy(rows1, acc_sh.at[dst_v.at[_GRP - 1]], ss1, add=True)
      wait(ss0, rows0)
      wait(ss1, rows1)
      return carry

    lax.fori_loop(0, ngroups, group, 0, unroll=False)
    plsc.subcore_barrier()

    pltpu.sync_copy(acc_sh.at[pl.ds(s * _DUMP, _DUMP)],
                    out_hbm.at[c].at[pl.ds(s * _DUMP, _DUMP)])

  return k(h, edges3, zeros_acc)


def _matmul_t(a, w):
  return lax.dot_general(a, w, (((1,), (1,)), ((), ())),
                         preferred_element_type=jnp.float32)


def _bn_relu(h, g, b):
  m = jnp.mean(h, axis=0, keepdims=True)
  v = jnp.mean((h - m) ** 2, axis=0, keepdims=True)
  return jnp.maximum(g * (h - m) * lax.rsqrt(v + 1e-5) + b, 0.0)


def _tc_in(x, W_in, b_in, g_in, beta_in):
  def body(x_ref, w_ref, b_ref, g_ref, be_ref, o_ref):
    h = _matmul_t(x_ref[...], w_ref[...]) + b_ref[...]
    o_ref[...] = _bn_relu(h, g_ref[...], be_ref[...])

  return pl.pallas_call(
      body,
      out_shape=jax.ShapeDtypeStruct((_N, _D), jnp.float32),
  )(x, W_in, b_in, g_in, beta_in)


def _tc_gin(h, acc, eps, Wa, ba, Wb, bb, g, beta):
  def body(h_ref, acc_ref, eps_ref, wa_ref, ba_ref, wb_ref, bb_ref, g_ref,
           be_ref, o_ref):
    z = ((2.0 + eps_ref[0, 0]) * h_ref[...] + acc_ref[0, :_N]
         + acc_ref[1, :_N])
    t = jnp.maximum(_matmul_t(z, wa_ref[...]) + ba_ref[...], 0.0)
    u = _matmul_t(t, wb_ref[...]) + bb_ref[...]
    o_ref[...] = _bn_relu(u, g_ref[...], be_ref[...])

  return pl.pallas_call(
      body,
      out_shape=jax.ShapeDtypeStruct((_N, _D), jnp.float32),
  )(h, acc, eps, Wa, ba, Wb, bb, g, beta)


def _tc_head(h, W_head, b_head):
  def body(h_ref, w_ref, b_ref, o_ref):
    o_ref[...] = _matmul_t(h_ref[...], w_ref[...]) + b_ref[...]

  return pl.pallas_call(
      body,
      out_shape=jax.ShapeDtypeStruct((_N, _D), jnp.float32),
  )(h, W_head, b_head)


def kernel(x, edge_index, eps1, eps2, W_in, b_in, g_in, beta_in, W1a, b1a,
           W1b, b1b, g1, beta1, W2a, b2a, W2b, b2b, g2, beta2, W_head,
           b_head):
  ei = edge_index.astype(jnp.int32)
  pad = _TOT - 2 * _E
  src = jnp.concatenate([ei[0], ei[1], jnp.zeros((pad,), jnp.int32)])
  pad_dst = _PAD_ROW + (jnp.arange(pad, dtype=jnp.int32) % (_ACC_ROWS - _N))
  dst = jnp.concatenate([ei[1], ei[0], pad_dst])
  edges3 = jnp.stack([src, dst]).reshape(2, _NROWS, _CHUNK)
  zeros_acc = jnp.zeros((_ACC_ROWS, _D), jnp.float32)

  r = lambda v: v.reshape(1, _D)
  x0 = _tc_in(x, W_in, r(b_in), r(g_in), r(beta_in))
  acc1 = _agg_sc(x0, edges3, zeros_acc)
  h1 = _tc_gin(x0, acc1, eps1.reshape(1, 1), W1a, r(b1a), W1b, r(b1b),
               r(g1), r(beta1))
  acc2 = _agg_sc(h1, edges3, zeros_acc)
  h2 = _tc_gin(h1, acc2, eps2.reshape(1, 1), W2a, r(b2a), W2b, r(b2b),
               r(g2), r(beta2))
  return _tc_head(h2, W_head, r(b_head))


# speedup vs baseline: 1.6472x; 1.0022x over previous
import functools

import jax
import jax.numpy as jnp
from jax import lax
from jax.experimental import pallas as pl
from jax.experimental.pallas import tpu as pltpu
from jax.experimental.pallas import tpu_sc as plsc

_N = 10000
_E = 320000
_D = 128

_NC = 2
_NS = 16
_CHUNK = 128
_R0 = 304
_R1 = 16
_NROWS = _NS * (_R0 + _R1)
_GRP = 16
_TOT = _NROWS * _CHUNK
_PAD_ROW = _N
_DUMP = 640
_ACC_ROWS = _NS * _DUMP


def _agg_sc(h, edges3, zeros_acc):
  mesh = plsc.VectorSubcoreMesh(core_axis_name="c", subcore_axis_name="s")

  @functools.partial(
      pl.kernel,
      out_type=jax.ShapeDtypeStruct((_NC, _ACC_ROWS, _D), jnp.float32),
      mesh=mesh,
      scratch_types=[
          pltpu.VMEM((_GRP, _CHUNK), jnp.int32),
          pltpu.VMEM((_GRP, _CHUNK), jnp.int32),
          pltpu.VMEM((_CHUNK, _D), jnp.float32),
          pltpu.VMEM((_CHUNK, _D), jnp.float32),
          pltpu.SemaphoreType.DMA,
          pltpu.SemaphoreType.DMA,
          pltpu.SemaphoreType.DMA,
          pltpu.SemaphoreType.DMA,
          pltpu.VMEM_SHARED((_ACC_ROWS, _D), jnp.float32),
      ],
  )
  def k(h_hbm, e_hbm, z_hbm, out_hbm, src_v, dst_v, rows0, rows1,
        gs0, gs1, ss0, ss1, acc_sh):
    c = lax.axis_index("c")
    s = lax.axis_index("s")
    row_base = jnp.where(c == 0, s * _R0, _NS * _R0 + s * _R1)
    ngroups = jnp.where(c == 0, _R0 // _GRP, _R1 // _GRP)

    @pl.when(s == 0)
    def _():
      pltpu.sync_copy(z_hbm, acc_sh)

    plsc.subcore_barrier()

    def wait(sem, buf):
      pltpu.make_async_copy(h_hbm.at[src_v.at[0]], buf, sem).wait()

    def group(g, carry):
      pltpu.sync_copy(e_hbm.at[0].at[pl.ds(row_base + g * _GRP, _GRP)],
                      src_v)
      pltpu.sync_copy(e_hbm.at[1].at[pl.ds(row_base + g * _GRP, _GRP)],
                      dst_v)

      pltpu.async_copy(h_hbm.at[src_v.at[0]], rows0, gs0)
      pltpu.async_copy(h_hbm.at[src_v.at[1]], rows1, gs1)

      def pair(i, c2):
        j0 = 2 * i
        wait(gs0, rows0)
        pltpu.async_copy(rows0, acc_sh.at[dst_v.at[j0]], ss0, add=True)
        wait(gs1, rows1)
        pltpu.async_copy(rows1, acc_sh.at[dst_v.at[j0 + 1]], ss1, add=True)
        wait(ss0, rows0)
        pltpu.async_copy(h_hbm.at[src_v.at[j0 + 2]], rows0, gs0)
        wait(ss1, rows1)
        pltpu.async_copy(h_hbm.at[src_v.at[j0 + 3]], rows1, gs1)
        return c2

      lax.fori_loop(0, _GRP // 2 - 1, pair, 0, unroll=False)

      wait(gs0, rows0)
      pltpu.async_copy(rows0, acc_sh.at[dst_v.at[_GRP - 2]], ss0, add=True)
      wait(gs1, rows1)
      pltpu.async_copy(rows1, acc_sh.at[dst_v.at[_GRP - 1]], ss1, add=True)
      wait(ss0, rows0)
      wait(ss1, rows1)
      return carry

    lax.fori_loop(0, ngroups, group, 0, unroll=False)
    plsc.subcore_barrier()

    pltpu.sync_copy(acc_sh.at[pl.ds(s * _DUMP, _DUMP)],
                    out_hbm.at[c].at[pl.ds(s * _DUMP, _DUMP)])

  return k(h, edges3, zeros_acc)


def _matmul_t(a, w):
  return lax.dot_general(a, w, (((1,), (1,)), ((), ())),
                         preferred_element_type=jnp.float32)


def _bn_relu(h, g, b):
  m = jnp.mean(h, axis=0, keepdims=True)
  v = jnp.mean((h - m) ** 2, axis=0, keepdims=True)
  return jnp.maximum(g * (h - m) * lax.rsqrt(v + 1e-5) + b, 0.0)


def _tc_in(x, W_in, b_in, g_in, beta_in):
  def body(x_ref, w_ref, b_ref, g_ref, be_ref, o_ref):
    h = _matmul_t(x_ref[...], w_ref[...]) + b_ref[...]
    o_ref[...] = _bn_relu(h, g_ref[...], be_ref[...])

  return pl.pallas_call(
      body,
      out_shape=jax.ShapeDtypeStruct((_N, _D), jnp.float32),
  )(x, W_in, b_in, g_in, beta_in)


def _tc_gin(h, acc, eps, Wa, ba, Wb, bb, g, beta, head=None):
  def body(h_ref, acc_ref, eps_ref, wa_ref, ba_ref, wb_ref, bb_ref, g_ref,
           be_ref, *rest):
    z = ((2.0 + eps_ref[0, 0]) * h_ref[...] + acc_ref[0, :_N]
         + acc_ref[1, :_N])
    t = jnp.maximum(_matmul_t(z, wa_ref[...]) + ba_ref[...], 0.0)
    u = _matmul_t(t, wb_ref[...]) + bb_ref[...]
    o = _bn_relu(u, g_ref[...], be_ref[...])
    if head is None:
      rest[-1][...] = o
    else:
      wh_ref, bh_ref = rest[0], rest[1]
      rest[-1][...] = _matmul_t(o, wh_ref[...]) + bh_ref[...]

  args = [h, acc, eps, Wa, ba, Wb, bb, g, beta]
  if head is not None:
    args += list(head)
  return pl.pallas_call(
      body,
      out_shape=jax.ShapeDtypeStruct((_N, _D), jnp.float32),
  )(*args)


def kernel(x, edge_index, eps1, eps2, W_in, b_in, g_in, beta_in, W1a, b1a,
           W1b, b1b, g1, beta1, W2a, b2a, W2b, b2b, g2, beta2, W_head,
           b_head):
  ei = edge_index.astype(jnp.int32)
  pad = _TOT - 2 * _E
  src = jnp.concatenate([ei[0], ei[1], jnp.zeros((pad,), jnp.int32)])
  pad_dst = _PAD_ROW + (jnp.arange(pad, dtype=jnp.int32) % (_ACC_ROWS - _N))
  dst = jnp.concatenate([ei[1], ei[0], pad_dst])
  edges3 = jnp.stack([src, dst]).reshape(2, _NROWS, _CHUNK)
  zeros_acc = jnp.zeros((_ACC_ROWS, _D), jnp.float32)

  r = lambda v: v.reshape(1, _D)
  x0 = _tc_in(x, W_in, r(b_in), r(g_in), r(beta_in))
  acc1 = _agg_sc(x0, edges3, zeros_acc)
  h1 = _tc_gin(x0, acc1, eps1.reshape(1, 1), W1a, r(b1a), W1b, r(b1b),
               r(g1), r(beta1))
  acc2 = _agg_sc(h1, edges3, zeros_acc)
  return _tc_gin(h1, acc2, eps2.reshape(1, 1), W2a, r(b2a), W2b, r(b2b),
                 r(g2), r(beta2), head=(W_head, r(b_head)))
